# Initial kernel scaffold; baseline (speedup 1.0000x reference)
import jax
import jax.numpy as jnp
from jax.experimental import pallas as pl


def kernel(position_ids, table):
    raise NotImplementedError("write your pallas kernel here")


# speedup vs baseline: 1.9853x; 1.9853x over previous
import functools

import jax
import jax.numpy as jnp
from jax import lax
from jax.experimental import pallas as pl
from jax.experimental.pallas import tpu as pltpu
from jax.experimental.pallas import tpu_sc as plsc


@functools.lru_cache(maxsize=None)
def _build(B, D):
    info = plsc.get_sparse_core_info()
    nw = info.num_cores * info.num_subcores
    assert B % (8 * nw) == 0
    b_per_w = B // nw
    C = 32
    nchunks = b_per_w // C
    mesh = plsc.VectorSubcoreMesh(core_axis_name="c", subcore_axis_name="s")

    @functools.partial(
        pl.kernel,
        mesh=mesh,
        out_type=jax.ShapeDtypeStruct((B, D), jnp.float32),
        scratch_types=[
            pltpu.VMEM((b_per_w,), jnp.int32),
            pltpu.VMEM((C, D), jnp.float32),
            pltpu.SemaphoreType.DMA,
        ],
    )
    def k(idx_hbm, table_hbm, out_hbm, idx_v, rows_v, gsem):
        wid = lax.axis_index("s") * info.num_cores + lax.axis_index("c")
        base = wid * b_per_w
        pltpu.sync_copy(idx_hbm.at[pl.ds(base, b_per_w)], idx_v)

        def body(c, carry):
            off = c * C
            pltpu.async_copy(
                table_hbm.at[idx_v.at[pl.ds(off, C)]], rows_v, gsem
            ).wait()
            pltpu.sync_copy(rows_v, out_hbm.at[pl.ds(base + off, C)])
            return carry

        lax.fori_loop(0, nchunks, body, 0)

    return k


def kernel(position_ids, table):
    bsz, seq = position_ids.shape
    d = table.shape[1]
    idx = position_ids.reshape(-1).astype(jnp.int32)
    out = _build(bsz * seq, d)(idx, table)
    return out.reshape(bsz, seq, d)


# speedup vs baseline: 2.2404x; 1.1285x over previous
import functools

import jax
import jax.numpy as jnp
from jax import lax
from jax.experimental import pallas as pl
from jax.experimental.pallas import tpu as pltpu
from jax.experimental.pallas import tpu_sc as plsc


@functools.lru_cache(maxsize=None)
def _build(B, D):
    info = plsc.get_sparse_core_info()
    nw = info.num_cores * info.num_subcores
    assert B % (8 * nw) == 0
    b_per_w = B // nw
    C = 32
    nchunks = b_per_w // C
    ngroups = nchunks // 2
    assert nchunks % 2 == 0 and ngroups >= 2
    mesh = plsc.VectorSubcoreMesh(core_axis_name="c", subcore_axis_name="s")

    @functools.partial(
        pl.kernel,
        mesh=mesh,
        out_type=jax.ShapeDtypeStruct((B, D), jnp.float32),
        scratch_types=[
            pltpu.VMEM((b_per_w,), jnp.int32),
            pltpu.VMEM((C, D), jnp.float32),
            pltpu.VMEM((C, D), jnp.float32),
            pltpu.SemaphoreType.DMA,
            pltpu.SemaphoreType.DMA,
            pltpu.SemaphoreType.DMA,
            pltpu.SemaphoreType.DMA,
        ],
    )
    def k(idx_hbm, table_hbm, out_hbm, idx_v, rows_a, rows_b,
          gsem_a, gsem_b, osem_a, osem_b):
        wid = lax.axis_index("s") * info.num_cores + lax.axis_index("c")
        base = wid * b_per_w
        pltpu.sync_copy(idx_hbm.at[pl.ds(base, b_per_w)], idx_v)

        def g_start(c, buf, sem):
            pltpu.async_copy(table_hbm.at[idx_v.at[pl.ds(c * C, C)]], buf, sem)

        def g_wait(buf, sem):
            pltpu.make_async_copy(
                table_hbm.at[idx_v.at[pl.ds(0, C)]], buf, sem).wait()

        def s_start(c, buf, sem):
            pltpu.async_copy(buf, out_hbm.at[pl.ds(base + c * C, C)], sem)

        def s_wait(buf, sem):
            pltpu.make_async_copy(
                buf, out_hbm.at[pl.ds(base, C)], sem).wait()

        def group(c0, first, last):
            if not first:
                s_wait(rows_b, osem_b)
            g_start(c0 + 1, rows_b, gsem_b)
            g_wait(rows_a, gsem_a)
            s_start(c0, rows_a, osem_a)
            g_wait(rows_b, gsem_b)
            s_start(c0 + 1, rows_b, osem_b)
            s_wait(rows_a, osem_a)
            if not last:
                g_start(c0 + 2, rows_a, gsem_a)

        g_start(0, rows_a, gsem_a)
        group(0, first=True, last=False)

        def body(g, carry):
            group(2 * g, first=False, last=False)
            return carry

        lax.fori_loop(1, ngroups - 1, body, 0)
        group(2 * (ngroups - 1), first=False, last=True)
        s_wait(rows_b, osem_b)

    return k


def kernel(position_ids, table):
    bsz, seq = position_ids.shape
    d = table.shape[1]
    idx = position_ids.reshape(-1).astype(jnp.int32)
    out = _build(bsz * seq, d)(idx, table)
    return out.reshape(bsz, seq, d)


# speedup vs baseline: 2.3844x; 1.0643x over previous
import functools

import jax
import jax.numpy as jnp
from jax import lax
from jax.experimental import pallas as pl
from jax.experimental.pallas import tpu as pltpu
from jax.experimental.pallas import tpu_sc as plsc

_NBUF = 3


@functools.lru_cache(maxsize=None)
def _build(B, D):
    info = plsc.get_sparse_core_info()
    nw = info.num_cores * info.num_subcores
    assert B % (8 * nw) == 0
    b_per_w = B // nw
    C = 32
    nchunks = b_per_w // C
    ngroups, rem = divmod(nchunks, _NBUF)
    assert ngroups >= 2
    mesh = plsc.VectorSubcoreMesh(core_axis_name="c", subcore_axis_name="s")

    @functools.partial(
        pl.kernel,
        mesh=mesh,
        out_type=jax.ShapeDtypeStruct((B, D), jnp.float32),
        scratch_types=[
            pltpu.VMEM((b_per_w,), jnp.int32),
            *[pltpu.VMEM((C, D), jnp.float32) for _ in range(_NBUF)],
            *[pltpu.SemaphoreType.DMA for _ in range(2 * _NBUF)],
        ],
    )
    def k(idx_hbm, table_hbm, out_hbm, idx_v, *scratch):
        bufs = scratch[:_NBUF]
        gsem = scratch[_NBUF:2 * _NBUF]
        osem = scratch[2 * _NBUF:]
        wid = lax.axis_index("s") * info.num_cores + lax.axis_index("c")
        base = wid * b_per_w
        pltpu.sync_copy(idx_hbm.at[pl.ds(base, b_per_w)], idx_v)

        def g_start(c, b):
            pltpu.async_copy(
                table_hbm.at[idx_v.at[pl.ds(c * C, C)]], bufs[b], gsem[b])

        def g_wait(b):
            pltpu.make_async_copy(
                table_hbm.at[idx_v.at[pl.ds(0, C)]], bufs[b], gsem[b]).wait()

        def s_start(c, b):
            pltpu.async_copy(bufs[b], out_hbm.at[pl.ds(base + c * C, C)],
                             osem[b])

        def s_wait(b):
            pltpu.make_async_copy(
                bufs[b], out_hbm.at[pl.ds(base, C)], osem[b]).wait()

        def step(c, b, head):
            if not head:
                s_wait(b)
            g_start(c, b)
            if not (head and b < 2):
                g_wait((b + 1) % _NBUF)
                s_start(c - 2, (b + 1) % _NBUF)

        for j in range(_NBUF):
            step(j, j, head=True)

        def body(g, carry):
            c0 = g * _NBUF
            for j in range(_NBUF):
                step(c0 + j, j, head=False)
            return carry

        lax.fori_loop(1, ngroups, body, 0)

        for j in range(rem):
            step(ngroups * _NBUF + j, j, head=False)

        for c in (nchunks - 2, nchunks - 1):
            g_wait(c % _NBUF)
            s_start(c, c % _NBUF)
        for b in range(_NBUF):
            s_wait(b)

    return k


def kernel(position_ids, table):
    bsz, seq = position_ids.shape
    d = table.shape[1]
    idx = position_ids.reshape(-1).astype(jnp.int32)
    out = _build(bsz * seq, d)(idx, table)
    return out.reshape(bsz, seq, d)
